# baseline (speedup 1.0000x reference)
import functools

import jax
import jax.numpy as jnp
from jax import lax
from jax.experimental import pallas as pl
from jax.experimental.pallas import tpu as pltpu
from jax.experimental.pallas import tpu_sc as plsc

_IDX_CHUNK = 128


@functools.partial(jax.jit, static_argnames=("b_per_w", "num_cores"))
def _sc_gather(cid, did, word_embeds, doc_weights, *, b_per_w, num_cores):
  B = cid.shape[0] * cid.shape[1]
  D = word_embeds.shape[1]
  T = doc_weights.shape[1]
  n_chunks = b_per_w // _IDX_CHUNK
  mesh = plsc.VectorSubcoreMesh(core_axis_name="c", subcore_axis_name="s")

  @functools.partial(
      pl.kernel,
      mesh=mesh,
      compiler_params=pltpu.CompilerParams(use_tc_tiling_on_sc=False),
      out_type=[
          jax.ShapeDtypeStruct((B, D), jnp.float32),
          jax.ShapeDtypeStruct((B, T), jnp.float32),
      ],
      scratch_types=[
          pltpu.VMEM((n_chunks, _IDX_CHUNK), jnp.int32),
          pltpu.VMEM((n_chunks, _IDX_CHUNK), jnp.int32),
          pltpu.VMEM((b_per_w, D), jnp.float32),
          pltpu.VMEM((b_per_w, T), jnp.float32),
          pltpu.SemaphoreType.DMA,
          pltpu.SemaphoreType.DMA,
      ],
  )
  def gather_k(cid_hbm, did_hbm, words_hbm, dw_hbm, wout_hbm, dwout_hbm,
               cidx_v, didx_v, wrows_v, dwrows_v, wsem, dsem):
    wid = lax.axis_index("s") * num_cores + lax.axis_index("c")
    base = wid * b_per_w
    pltpu.sync_copy(cid_hbm.at[pl.ds(wid * n_chunks, n_chunks)], cidx_v)
    pltpu.sync_copy(did_hbm.at[pl.ds(wid * n_chunks, n_chunks)], didx_v)
    copies = []
    for j in range(n_chunks):
      copies.append(pltpu.async_copy(
          words_hbm.at[cidx_v.at[j]],
          wrows_v.at[pl.ds(j * _IDX_CHUNK, _IDX_CHUNK)], wsem))
      copies.append(pltpu.async_copy(
          dw_hbm.at[didx_v.at[j]],
          dwrows_v.at[pl.ds(j * _IDX_CHUNK, _IDX_CHUNK)], dsem))
    for c in copies:
      c.wait()
    pltpu.sync_copy(wrows_v, wout_hbm.at[pl.ds(base, b_per_w)])
    pltpu.sync_copy(dwrows_v, dwout_hbm.at[pl.ds(base, b_per_w)])

  return gather_k(cid, did, word_embeds, doc_weights)


def _tc_combine(word_vecs, dw, topic_embeds, *, block_b=2048):
  B, D = word_vecs.shape
  T = dw.shape[1]

  def body(w_ref, dw_ref, t_ref, o_ref):
    dwb = dw_ref[...]
    m = jnp.max(dwb, axis=1, keepdims=True)
    e = jnp.exp(dwb - m)
    s = jnp.sum(e, axis=1, keepdims=True)
    doc = jnp.dot(e, t_ref[...], preferred_element_type=jnp.float32) / s
    o_ref[...] = w_ref[...] + doc

  return pl.pallas_call(
      body,
      grid=(B // block_b,),
      in_specs=[
          pl.BlockSpec((block_b, D), lambda i: (i, 0)),
          pl.BlockSpec((block_b, T), lambda i: (i, 0)),
          pl.BlockSpec((T, D), lambda i: (0, 0)),
      ],
      out_specs=pl.BlockSpec((block_b, D), lambda i: (i, 0)),
      out_shape=jax.ShapeDtypeStruct((B, D), jnp.float32),
  )(word_vecs, dw, topic_embeds)


def kernel(center_id, doc_id, word_embeds, doc_weights, topic_embeds):
  B = center_id.shape[0]
  info = plsc.get_sparse_core_info()
  nw = info.num_cores * info.num_subcores
  b_per_w = B // nw
  cid = center_id.reshape(B // _IDX_CHUNK, _IDX_CHUNK).astype(jnp.int32)
  did = doc_id.reshape(B // _IDX_CHUNK, _IDX_CHUNK).astype(jnp.int32)
  word_vecs, dw = _sc_gather(cid, did, word_embeds, doc_weights,
                             b_per_w=b_per_w, num_cores=info.num_cores)
  return _tc_combine(word_vecs, dw, topic_embeds)
